# baseline (speedup 1.0000x reference)
import functools

import jax
import jax.numpy as jnp
from jax import lax
from jax.experimental import pallas as pl
from jax.experimental.pallas import tpu as pltpu
from jax.experimental.pallas import tpu_sc as plsc

_BATCH = 1024
_HIDDEN = 64
_VOCAB = 100000
_VT = 2048

_NC = 2
_NS = 16
_NW = _NC * _NS
_BPW = _BATCH // _NW


def _gather_sc(table, ids):
    mesh = plsc.ScalarSubcoreMesh(axis_name="c", num_cores=_NC)
    per_core = _BATCH // _NC
    chunk = 16

    @functools.partial(
        pl.kernel,
        mesh=mesh,
        out_type=jax.ShapeDtypeStruct((_BATCH, _HIDDEN), jnp.float32),
        scratch_types=[
            pltpu.SMEM((per_core,), jnp.int32),
            pltpu.SemaphoreType.DMA,
            pltpu.SemaphoreType.DMA,
        ],
    )
    def k(table_hbm, idx_hbm, out_hbm, idx_s, sem_in, sem_out):
        cid = lax.axis_index("c")
        base = cid * per_core
        pltpu.async_copy(idx_hbm.at[pl.ds(base, per_core)], idx_s, sem_in).wait()

        @pl.loop(0, per_core, step=chunk)
        def _(c):
            handles = []
            for j in range(chunk):
                rid = idx_s[c + j]
                handles.append(
                    pltpu.async_copy(
                        table_hbm.at[pl.ds(rid, 1)],
                        out_hbm.at[pl.ds(base + c + j, 1)],
                        sem_out,
                    )
                )
            for h in handles:
                h.wait()

    return k(table, ids)


def _project_body(emb_ref, w_ref, b_ref, out_ref):
    acc = lax.dot_general(
        emb_ref[...],
        w_ref[...],
        (((1,), (1,)), ((), ())),
        preferred_element_type=jnp.float32,
    )
    out_ref[...] = acc + b_ref[...]


def _project(emb, weight, bias2d, interpret=False):
    return pl.pallas_call(
        _project_body,
        grid=(pl.cdiv(_VOCAB, _VT),),
        in_specs=[
            pl.BlockSpec((_BATCH, _HIDDEN), lambda i: (0, 0)),
            pl.BlockSpec((_VT, _HIDDEN), lambda i: (i, 0)),
            pl.BlockSpec((1, _VT), lambda i: (0, i)),
        ],
        out_specs=pl.BlockSpec((_BATCH, _VT), lambda i: (0, i)),
        out_shape=jax.ShapeDtypeStruct((_BATCH, _VOCAB), jnp.float32),
        interpret=interpret,
    )(emb, weight, bias2d)


def kernel(input_ids, embedding_weight, linear_weight, linear_bias):
    ids = input_ids.astype(jnp.int32)
    emb = _gather_sc(embedding_weight, ids)
    bias2d = linear_bias.reshape(1, _VOCAB)
    return _project(emb, linear_weight, bias2d)


# speedup vs baseline: 1.0183x; 1.0183x over previous
import functools

import jax
import jax.numpy as jnp
from jax import lax
from jax.experimental import pallas as pl
from jax.experimental.pallas import tpu as pltpu
from jax.experimental.pallas import tpu_sc as plsc

_BATCH = 1024
_HIDDEN = 64
_VOCAB = 100000
_VT = 2048

_NC = 2
_NS = 16
_NW = _NC * _NS
_BPW = _BATCH // _NW


def _gather_sc(table, ids):
    mesh = plsc.ScalarSubcoreMesh(axis_name="c", num_cores=_NC)
    per_core = _BATCH // _NC
    chunk = 16

    @functools.partial(
        pl.kernel,
        mesh=mesh,
        out_type=jax.ShapeDtypeStruct((_BATCH, _HIDDEN), jnp.float32),
        scratch_types=[
            pltpu.SMEM((per_core,), jnp.int32),
            pltpu.SemaphoreType.DMA,
            pltpu.SemaphoreType.DMA,
        ],
    )
    def k(table_hbm, idx_hbm, out_hbm, idx_s, sem_in, sem_out):
        cid = lax.axis_index("c")
        base = cid * per_core
        pltpu.async_copy(idx_hbm.at[pl.ds(base, per_core)], idx_s, sem_in).wait()

        @pl.loop(0, per_core, step=chunk)
        def _(c):
            for j in range(chunk):
                rid = idx_s[c + j]
                pltpu.async_copy(
                    table_hbm.at[pl.ds(rid, 1)],
                    out_hbm.at[pl.ds(base + c + j, 1)],
                    sem_out,
                )

        @pl.loop(0, per_core, step=chunk)
        def _(c):
            for j in range(chunk):
                pltpu.make_async_copy(
                    table_hbm.at[pl.ds(0, 1)],
                    out_hbm.at[pl.ds(base + c + j, 1)],
                    sem_out,
                ).wait()

    return k(table, ids)


def _project_body(emb_ref, w_ref, b_ref, out_ref):
    acc = lax.dot_general(
        emb_ref[...].astype(jnp.bfloat16),
        w_ref[...].astype(jnp.bfloat16),
        (((1,), (1,)), ((), ())),
        preferred_element_type=jnp.float32,
    )
    out_ref[...] = acc + b_ref[...]


def _project(emb, weight, bias2d, interpret=False):
    return pl.pallas_call(
        _project_body,
        grid=(pl.cdiv(_VOCAB, _VT),),
        in_specs=[
            pl.BlockSpec((_BATCH, _HIDDEN), lambda i: (0, 0)),
            pl.BlockSpec((_VT, _HIDDEN), lambda i: (i, 0)),
            pl.BlockSpec((1, _VT), lambda i: (0, i)),
        ],
        out_specs=pl.BlockSpec((_BATCH, _VT), lambda i: (0, i)),
        out_shape=jax.ShapeDtypeStruct((_BATCH, _VOCAB), jnp.float32),
        interpret=interpret,
    )(emb, weight, bias2d)


def kernel(input_ids, embedding_weight, linear_weight, linear_bias):
    ids = input_ids.astype(jnp.int32)
    emb = _gather_sc(embedding_weight, ids)
    bias2d = linear_bias.reshape(1, _VOCAB)
    return _project(emb, linear_weight, bias2d)


# speedup vs baseline: 2.9393x; 2.8865x over previous
import functools

import jax
import jax.numpy as jnp
from jax import lax
from jax.experimental import pallas as pl
from jax.experimental.pallas import tpu as pltpu
from jax.experimental.pallas import tpu_sc as plsc

_BATCH = 1024
_HIDDEN = 64
_VOCAB = 100000
_VT = 2048

_NC = 2
_NS = 16
_NW = _NC * _NS
_BPW = _BATCH // _NW


def _gather_sc(table, ids):
    mesh = plsc.ScalarSubcoreMesh(axis_name="c", num_cores=_NC)
    per_core = _BATCH // _NC
    chunk = 16

    @functools.partial(
        pl.kernel,
        mesh=mesh,
        out_type=jax.ShapeDtypeStruct((_BATCH, _HIDDEN), jnp.float32),
        scratch_types=[
            pltpu.SMEM((per_core,), jnp.int32),
            pltpu.SemaphoreType.DMA,
            pltpu.SemaphoreType.DMA,
        ],
    )
    def k(table_hbm, idx_hbm, out_hbm, idx_s, sem_in, sem_out):
        cid = lax.axis_index("c")
        base = cid * per_core
        pltpu.async_copy(idx_hbm.at[pl.ds(base, per_core)], idx_s, sem_in).wait()

        @pl.loop(0, per_core, step=chunk)
        def _(c):
            for j in range(chunk):
                rid = idx_s[c + j]
                pltpu.async_copy(
                    table_hbm.at[pl.ds(rid, 1)],
                    out_hbm.at[pl.ds(base + c + j, 1)],
                    sem_out,
                )

        @pl.loop(0, per_core, step=chunk)
        def _(c):
            for j in range(chunk):
                pltpu.make_async_copy(
                    table_hbm.at[pl.ds(0, 1)],
                    out_hbm.at[pl.ds(base + c + j, 1)],
                    sem_out,
                ).wait()

    return k(table, ids)


def _project_body(wT_ref, emb_ref, b_ref, out_ref):
    acc = lax.dot_general(
        wT_ref[...].astype(jnp.bfloat16),
        emb_ref[...].astype(jnp.bfloat16),
        (((0,), (1,)), ((), ())),
        preferred_element_type=jnp.float32,
    )
    out_ref[...] = acc + jnp.transpose(b_ref[...])


def _project(wT, emb, bias2d, interpret=False):
    return pl.pallas_call(
        _project_body,
        grid=(pl.cdiv(_VOCAB, _VT),),
        in_specs=[
            pl.BlockSpec((_HIDDEN, _VT), lambda i: (0, i)),
            pl.BlockSpec((_BATCH, _HIDDEN), lambda i: (0, 0)),
            pl.BlockSpec((1, _VT), lambda i: (0, i)),
        ],
        out_specs=pl.BlockSpec((_VT, _BATCH), lambda i: (i, 0)),
        out_shape=jax.ShapeDtypeStruct((_VOCAB, _BATCH), jnp.float32),
        interpret=interpret,
    )(wT, emb, bias2d)


def kernel(input_ids, embedding_weight, linear_weight, linear_bias):
    ids = input_ids.astype(jnp.int32)
    emb = _gather_sc(embedding_weight, ids)
    bias2d = linear_bias.reshape(1, _VOCAB)
    logitsT = _project(linear_weight.T, emb, bias2d)
    return logitsT.T
